# baseline (speedup 1.0000x reference)
import jax
import jax.numpy as jnp
from jax import lax
from jax.experimental import pallas as pl
from jax.experimental.pallas import tpu as pltpu
from jax.experimental.pallas import tpu_sc as plsc

N = 10000
E = 320000
DIN = 128
DH = 256
G = 64

NT = 10240
EP = 327680
TILES = 16
CH = 128
EPT = EP // TILES
NCH = EPT // CH
RPT = NT // TILES
GT = 2 * G
DP = 128

_f32 = jnp.float32

_mesh = plsc.VectorSubcoreMesh(core_axis_name="c", subcore_axis_name="s")


def _sc_edges(x_hbm, ei_hbm, table, islot0, islot1, rows0, rows1,
              gsem0, gsem1, ssem0, ssem1, row0):
  pltpu.sync_copy(ei_hbm.at[row0], islot0)
  pltpu.async_copy(x_hbm.at[islot0.at[0, 0]], rows0, gsem0)

  def body(k, carry):
    r0 = row0 + 2 * k
    @pl.when(k > 0)
    def _():
      pltpu.make_async_copy(rows1, table.at[islot1.at[1, 0]], ssem1).wait()

    pltpu.sync_copy(ei_hbm.at[r0 + 1], islot1)
    g1 = pltpu.async_copy(x_hbm.at[islot1.at[0, 0]], rows1, gsem1)
    pltpu.make_async_copy(x_hbm.at[islot0.at[0, 0]], rows0, gsem0).wait()
    s0 = pltpu.async_copy(rows0, table.at[islot0.at[1, 0]], ssem0, add=True)
    g1.wait()
    pltpu.async_copy(rows1, table.at[islot1.at[1, 0]], ssem1, add=True)

    @pl.when(k + 1 < NCH // 2)
    def _():
      s0.wait()
      pltpu.sync_copy(ei_hbm.at[r0 + 2], islot0)
      pltpu.async_copy(x_hbm.at[islot0.at[0, 0]], rows0, gsem0)

    return carry

  lax.fori_loop(0, NCH // 2, body, 0)
  pltpu.make_async_copy(rows0, table.at[islot0.at[1, 0]], ssem0).wait()
  pltpu.make_async_copy(rows1, table.at[islot1.at[1, 0]], ssem1).wait()


def _sc_conv_body(xa, xb, eia, eib, outa, outb,
                  table, islot0, islot1, rows0, rows1,
                  gsem0, gsem1, ssem0, ssem1):
  c = lax.axis_index("c")
  s = lax.axis_index("s")

  def run(x_hbm, ei_hbm, out_hbm):
    pltpu.sync_copy(x_hbm.at[pl.ds(s * RPT, RPT)],
                    table.at[pl.ds(s * RPT, RPT)])
    plsc.subcore_barrier()
    _sc_edges(x_hbm, ei_hbm, table, islot0, islot1, rows0, rows1,
              gsem0, gsem1, ssem0, ssem1, s * NCH)
    plsc.subcore_barrier()
    pltpu.sync_copy(table.at[pl.ds(s * RPT, RPT)],
                    out_hbm.at[pl.ds(s * RPT, RPT)])

  @pl.when(c == 0)
  def _():
    run(xa, eia, outa)

  @pl.when(c == 1)
  def _():
    run(xb, eib, outb)


_sc_conv = pl.kernel(
    _sc_conv_body,
    out_type=(jax.ShapeDtypeStruct((NT, DP), _f32),
              jax.ShapeDtypeStruct((NT, DP), _f32)),
    mesh=_mesh,
    scratch_types=[
        pltpu.VMEM_SHARED((NT, DP), _f32),
        pltpu.VMEM((2, 1, CH), jnp.int32),
        pltpu.VMEM((2, 1, CH), jnp.int32),
        pltpu.VMEM((CH, DP), _f32),
        pltpu.VMEM((CH, DP), _f32),
        pltpu.SemaphoreType.DMA,
        pltpu.SemaphoreType.DMA,
        pltpu.SemaphoreType.DMA,
        pltpu.SemaphoreType.DMA,
    ],
)


_RB = 1024
_HI = lax.Precision.HIGHEST


def _mlp1_body(x_ref, w1_ref, b1_ref, w2_ref, b2_ref, out_ref):
  h = jnp.dot(x_ref[...], w1_ref[...], preferred_element_type=_f32,
              precision=_HI)
  h = jnp.maximum(h + b1_ref[...], 0.0)
  out_ref[...] = jnp.dot(h, w2_ref[...], preferred_element_type=_f32,
                         precision=_HI) + b2_ref[...]


def _mlp1(x, w1, b1, w2, b2):
  rows, din = x.shape
  return pl.pallas_call(
      _mlp1_body,
      grid=(rows // _RB,),
      in_specs=[
          pl.BlockSpec((_RB, din), lambda i: (i, 0)),
          pl.BlockSpec((din, DH), lambda i: (0, 0)),
          pl.BlockSpec((1, DH), lambda i: (0, 0)),
          pl.BlockSpec((DH, DH), lambda i: (0, 0)),
          pl.BlockSpec((1, DH), lambda i: (0, 0)),
      ],
      out_specs=pl.BlockSpec((_RB, DH), lambda i: (i, 0)),
      out_shape=jax.ShapeDtypeStruct((rows, DH), _f32),
  )(x, w1, b1.reshape(1, DH), w2, b2.reshape(1, DH))


def _mlp2_pool_body(x_ref, seg_ref, w1_ref, b1_ref, w2_ref, b2_ref, out_ref):
  i = pl.program_id(0)
  h = jnp.dot(x_ref[...], w1_ref[...], preferred_element_type=_f32,
              precision=_HI)
  h = jnp.maximum(h + b1_ref[...], 0.0)
  h = jnp.dot(h, w2_ref[...], preferred_element_type=_f32,
              precision=_HI) + b2_ref[...]
  seg = seg_ref[0]
  ids = lax.broadcasted_iota(jnp.int32, (GT, _RB), 0)
  onehot = (ids == seg).astype(_f32)
  contrib = jnp.dot(onehot, h, preferred_element_type=_f32, precision=_HI)

  @pl.when(i == 0)
  def _():
    out_ref[...] = jnp.zeros_like(out_ref)

  out_ref[...] += contrib


def _mlp2_pool(x, seg3d, w1, b1, w2, b2):
  rows = x.shape[0]
  return pl.pallas_call(
      _mlp2_pool_body,
      grid=(rows // _RB,),
      in_specs=[
          pl.BlockSpec((_RB, DH), lambda i: (i, 0)),
          pl.BlockSpec((1, 1, _RB), lambda i: (i, 0, 0)),
          pl.BlockSpec((DH, DH), lambda i: (0, 0)),
          pl.BlockSpec((1, DH), lambda i: (0, 0)),
          pl.BlockSpec((DH, DH), lambda i: (0, 0)),
          pl.BlockSpec((1, DH), lambda i: (0, 0)),
      ],
      out_specs=pl.BlockSpec((GT, DH), lambda i: (0, 0)),
      out_shape=jax.ShapeDtypeStruct((GT, DH), _f32),
  )(x, seg3d, w1, b1.reshape(1, DH), w2, b2.reshape(1, DH))


def _head_body(p_ref, w1_ref, b1_ref, w2_ref, b2_ref, out_ref):
  p1 = p_ref[:G, :]
  p2 = p_ref[G:, :]
  xcat = jnp.concatenate([p1, p2], axis=1)
  h = jnp.dot(xcat, w1_ref[...], preferred_element_type=_f32, precision=_HI)
  h = jnp.maximum(h + b1_ref[...], 0.0)
  o = jnp.dot(h, w2_ref[...], preferred_element_type=_f32,
              precision=_HI) + b2_ref[...]
  out_ref[...] = 1.0 / (1.0 + jnp.exp(-o))


def _head(p, fc1_W, fc1_b, fc2_W, fc2_b):
  w2p = jnp.pad(fc2_W, ((0, 0), (0, 127)))
  b2p = jnp.pad(fc2_b.reshape(1, 1), ((0, 0), (0, 127)))
  out = pl.pallas_call(
      _head_body,
      out_shape=jax.ShapeDtypeStruct((G, 128), _f32),
  )(p, fc1_W, fc1_b.reshape(1, DH), w2p, b2p)
  return out[:, :1]


@jax.jit
def kernel(x1, edge_index1, batch1, x2, edge_index2, batch2,
           gin1_W1, gin1_b1, gin1_W2, gin1_b2,
           gin2_W1, gin2_b1, gin2_W2, gin2_b2,
           fc1_W, fc1_b, fc2_W, fc2_b):
  x1p = jnp.pad(x1, ((0, NT - N), (0, 0)))
  x2p = jnp.pad(x2, ((0, NT - N), (0, 0)))

  def _prep(ei):
    p = jnp.pad(ei, ((0, 0), (0, EP - E)), constant_values=N)
    return p.reshape(2, EP // CH, 1, CH).transpose(1, 0, 2, 3)

  e1 = _prep(edge_index1)
  e2 = _prep(edge_index2)
  seg1 = jnp.pad(batch1, (0, NT - N), constant_values=GT)
  seg2 = jnp.pad(batch2 + G, (0, NT - N), constant_values=GT)
  seg_all = jnp.concatenate([seg1, seg2]).reshape(2 * NT // _RB, 1, _RB)

  o1, o2 = _sc_conv(x1p, x2p, e1, e2)
  h_all = _mlp1(jnp.concatenate([o1, o2], axis=0),
                gin1_W1, gin1_b1, gin1_W2, gin1_b2)

  h1 = h_all[:NT]
  h2 = h_all[NT:]
  g1a, g1b = _sc_conv(h1[:, :DP], h1[:, DP:], e1, e1)
  g2a, g2b = _sc_conv(h2[:, :DP], h2[:, DP:], e2, e2)
  x2all = jnp.concatenate(
      [jnp.concatenate([g1a, g1b], axis=1),
       jnp.concatenate([g2a, g2b], axis=1)], axis=0)
  pooled = _mlp2_pool(x2all, seg_all,
                      gin2_W1, gin2_b1, gin2_W2, gin2_b2)

  return _head(pooled, fc1_W, fc1_b, fc2_W, fc2_b)


# speedup vs baseline: 1.0819x; 1.0819x over previous
import jax
import jax.numpy as jnp
from jax import lax
from jax.experimental import pallas as pl
from jax.experimental.pallas import tpu as pltpu
from jax.experimental.pallas import tpu_sc as plsc

N = 10000
E = 320000
DIN = 128
DH = 256
G = 64

NT = 10240
EP = 327680
TILES = 16
CH = 128
EPT = EP // TILES
NCH = EPT // CH
RPT = NT // TILES
GT = 2 * G
DP = 128

_f32 = jnp.float32

_mesh = plsc.VectorSubcoreMesh(core_axis_name="c", subcore_axis_name="s")


def _sc_edges(x_hbm, ei_hbm, table, islot0, islot1, rows0, rows1,
              gsem0, gsem1, ssem0, ssem1, row0):
  pltpu.sync_copy(ei_hbm.at[row0], islot0)
  pltpu.async_copy(x_hbm.at[islot0.at[0, 0]], rows0, gsem0)

  def body(k, carry):
    r0 = row0 + 2 * k
    @pl.when(k > 0)
    def _():
      pltpu.make_async_copy(rows1, table.at[islot1.at[1, 0]], ssem1).wait()

    pltpu.sync_copy(ei_hbm.at[r0 + 1], islot1)
    g1 = pltpu.async_copy(x_hbm.at[islot1.at[0, 0]], rows1, gsem1)
    pltpu.make_async_copy(x_hbm.at[islot0.at[0, 0]], rows0, gsem0).wait()
    s0 = pltpu.async_copy(rows0, table.at[islot0.at[1, 0]], ssem0, add=True)

    @pl.when(k + 1 < NCH // 2)
    def _():
      s0.wait()
      pltpu.sync_copy(ei_hbm.at[r0 + 2], islot0)
      pltpu.async_copy(x_hbm.at[islot0.at[0, 0]], rows0, gsem0)

    g1.wait()
    pltpu.async_copy(rows1, table.at[islot1.at[1, 0]], ssem1, add=True)
    return carry

  lax.fori_loop(0, NCH // 2, body, 0)
  pltpu.make_async_copy(rows0, table.at[islot0.at[1, 0]], ssem0).wait()
  pltpu.make_async_copy(rows1, table.at[islot1.at[1, 0]], ssem1).wait()


def _sc_conv_body(xa, xb, eia, eib, outa, outb,
                  table, islot0, islot1, rows0, rows1,
                  gsem0, gsem1, ssem0, ssem1):
  c = lax.axis_index("c")
  s = lax.axis_index("s")

  def run(x_hbm, ei_hbm, out_hbm):
    pltpu.sync_copy(x_hbm.at[pl.ds(s * RPT, RPT)],
                    table.at[pl.ds(s * RPT, RPT)])
    plsc.subcore_barrier()
    _sc_edges(x_hbm, ei_hbm, table, islot0, islot1, rows0, rows1,
              gsem0, gsem1, ssem0, ssem1, s * NCH)
    plsc.subcore_barrier()
    pltpu.sync_copy(table.at[pl.ds(s * RPT, RPT)],
                    out_hbm.at[pl.ds(s * RPT, RPT)])

  @pl.when(c == 0)
  def _():
    run(xa, eia, outa)

  @pl.when(c == 1)
  def _():
    run(xb, eib, outb)


_sc_conv = pl.kernel(
    _sc_conv_body,
    out_type=(jax.ShapeDtypeStruct((NT, DP), _f32),
              jax.ShapeDtypeStruct((NT, DP), _f32)),
    mesh=_mesh,
    scratch_types=[
        pltpu.VMEM_SHARED((NT, DP), _f32),
        pltpu.VMEM((2, 1, CH), jnp.int32),
        pltpu.VMEM((2, 1, CH), jnp.int32),
        pltpu.VMEM((CH, DP), _f32),
        pltpu.VMEM((CH, DP), _f32),
        pltpu.SemaphoreType.DMA,
        pltpu.SemaphoreType.DMA,
        pltpu.SemaphoreType.DMA,
        pltpu.SemaphoreType.DMA,
    ],
)


_RB = 1024
_HI = lax.Precision.HIGHEST


def _mlp1_body(x_ref, w1_ref, b1_ref, w2_ref, b2_ref, out_ref):
  h = jnp.dot(x_ref[...], w1_ref[...], preferred_element_type=_f32,
              precision=_HI)
  h = jnp.maximum(h + b1_ref[...], 0.0)
  out_ref[...] = jnp.dot(h, w2_ref[...], preferred_element_type=_f32,
                         precision=_HI) + b2_ref[...]


def _mlp1(x, w1, b1, w2, b2):
  rows, din = x.shape
  return pl.pallas_call(
      _mlp1_body,
      grid=(rows // _RB,),
      in_specs=[
          pl.BlockSpec((_RB, din), lambda i: (i, 0)),
          pl.BlockSpec((din, DH), lambda i: (0, 0)),
          pl.BlockSpec((1, DH), lambda i: (0, 0)),
          pl.BlockSpec((DH, DH), lambda i: (0, 0)),
          pl.BlockSpec((1, DH), lambda i: (0, 0)),
      ],
      out_specs=pl.BlockSpec((_RB, DH), lambda i: (i, 0)),
      out_shape=jax.ShapeDtypeStruct((rows, DH), _f32),
  )(x, w1, b1.reshape(1, DH), w2, b2.reshape(1, DH))


def _mlp2_pool_body(x_ref, seg_ref, w1_ref, b1_ref, w2_ref, b2_ref, out_ref):
  i = pl.program_id(0)
  h = jnp.dot(x_ref[...], w1_ref[...], preferred_element_type=_f32,
              precision=_HI)
  h = jnp.maximum(h + b1_ref[...], 0.0)
  h = jnp.dot(h, w2_ref[...], preferred_element_type=_f32,
              precision=_HI) + b2_ref[...]
  seg = seg_ref[0]
  ids = lax.broadcasted_iota(jnp.int32, (GT, _RB), 0)
  onehot = (ids == seg).astype(_f32)
  contrib = jnp.dot(onehot, h, preferred_element_type=_f32, precision=_HI)

  @pl.when(i == 0)
  def _():
    out_ref[...] = jnp.zeros_like(out_ref)

  out_ref[...] += contrib


def _mlp2_pool(x, seg3d, w1, b1, w2, b2):
  rows = x.shape[0]
  return pl.pallas_call(
      _mlp2_pool_body,
      grid=(rows // _RB,),
      in_specs=[
          pl.BlockSpec((_RB, DH), lambda i: (i, 0)),
          pl.BlockSpec((1, 1, _RB), lambda i: (i, 0, 0)),
          pl.BlockSpec((DH, DH), lambda i: (0, 0)),
          pl.BlockSpec((1, DH), lambda i: (0, 0)),
          pl.BlockSpec((DH, DH), lambda i: (0, 0)),
          pl.BlockSpec((1, DH), lambda i: (0, 0)),
      ],
      out_specs=pl.BlockSpec((GT, DH), lambda i: (0, 0)),
      out_shape=jax.ShapeDtypeStruct((GT, DH), _f32),
  )(x, seg3d, w1, b1.reshape(1, DH), w2, b2.reshape(1, DH))


def _head_body(p_ref, w1_ref, b1_ref, w2_ref, b2_ref, out_ref):
  p1 = p_ref[:G, :]
  p2 = p_ref[G:, :]
  xcat = jnp.concatenate([p1, p2], axis=1)
  h = jnp.dot(xcat, w1_ref[...], preferred_element_type=_f32, precision=_HI)
  h = jnp.maximum(h + b1_ref[...], 0.0)
  o = jnp.dot(h, w2_ref[...], preferred_element_type=_f32,
              precision=_HI) + b2_ref[...]
  out_ref[...] = 1.0 / (1.0 + jnp.exp(-o))


def _head(p, fc1_W, fc1_b, fc2_W, fc2_b):
  w2p = jnp.pad(fc2_W, ((0, 0), (0, 127)))
  b2p = jnp.pad(fc2_b.reshape(1, 1), ((0, 0), (0, 127)))
  out = pl.pallas_call(
      _head_body,
      out_shape=jax.ShapeDtypeStruct((G, 128), _f32),
  )(p, fc1_W, fc1_b.reshape(1, DH), w2p, b2p)
  return out[:, :1]


@jax.jit
def kernel(x1, edge_index1, batch1, x2, edge_index2, batch2,
           gin1_W1, gin1_b1, gin1_W2, gin1_b2,
           gin2_W1, gin2_b1, gin2_W2, gin2_b2,
           fc1_W, fc1_b, fc2_W, fc2_b):
  x1p = jnp.pad(x1, ((0, NT - N), (0, 0)))
  x2p = jnp.pad(x2, ((0, NT - N), (0, 0)))

  def _prep(ei):
    p = jnp.pad(ei, ((0, 0), (0, EP - E)), constant_values=N)
    return p.reshape(2, EP // CH, 1, CH).transpose(1, 0, 2, 3)

  e1 = _prep(edge_index1)
  e2 = _prep(edge_index2)
  seg1 = jnp.pad(batch1, (0, NT - N), constant_values=GT)
  seg2 = jnp.pad(batch2 + G, (0, NT - N), constant_values=GT)
  seg_all = jnp.concatenate([seg1, seg2]).reshape(2 * NT // _RB, 1, _RB)

  o1, o2 = _sc_conv(x1p, x2p, e1, e2)
  hx1 = _mlp1(o1, gin1_W1, gin1_b1, gin1_W2, gin1_b2)
  g1a, g1b = _sc_conv(hx1[:, :DP], hx1[:, DP:], e1, e1)
  hx2 = _mlp1(o2, gin1_W1, gin1_b1, gin1_W2, gin1_b2)
  g2a, g2b = _sc_conv(hx2[:, :DP], hx2[:, DP:], e2, e2)
  pooled1 = _mlp2_pool(jnp.concatenate([g1a, g1b], axis=1),
                       seg_all[:NT // _RB],
                       gin2_W1, gin2_b1, gin2_W2, gin2_b2)
  pooled2 = _mlp2_pool(jnp.concatenate([g2a, g2b], axis=1),
                       seg_all[NT // _RB:],
                       gin2_W1, gin2_b1, gin2_W2, gin2_b2)
  pooled = pooled1 + pooled2

  return _head(pooled, fc1_W, fc1_b, fc2_W, fc2_b)
